# Initial kernel scaffold; baseline (speedup 1.0000x reference)
import jax
import jax.numpy as jnp
from jax.experimental import pallas as pl


def kernel(social_edges, rates_edges, rated_edges, u2e_w, v2e_w, sage_social_ws, sage_social_wn, sage_social_b, sage_rates_ws, sage_rates_wn, sage_rates_b, sage_rated_ws, sage_rated_wn, sage_rated_b, wur1_w, wur1_b, wur2_w, wur2_b, wvr1_w, wvr1_b, wvr2_w, wvr2_b, wuv1_w, wuv1_b, wuv2_w, wuv2_b, wuv3_w, wuv3_b, bn1_g, bn1_b, bn2_g, bn2_b, bn3_g, bn3_b, bn4_g, bn4_b):
    raise NotImplementedError("write your pallas kernel here")


# speedup vs baseline: 3.4012x; 3.4012x over previous
import functools

import jax
import jax.numpy as jnp
from jax import lax
from jax.experimental import pallas as pl
from jax.experimental.pallas import tpu as pltpu
from jax.experimental.pallas import tpu_sc as plsc

NU = 50000
NI = 50000
D = 64
E = 800000
EPS = 1e-5

NC = 2
NS = 16
CW = 128
EP = 819200
R = EP // CW
RPT = R // NS
SEG_CH = 4
SEG_IT = RPT // SEG_CH
NROW = 3128
NUP = NROW * NS
PAD_DST = NU

f32 = jnp.float32
i32 = jnp.int32

_mesh = plsc.VectorSubcoreMesh(core_axis_name="c", subcore_axis_name="s",
                               num_cores=NC, num_subcores=NS)
_sc_params = pltpu.CompilerParams(use_tc_tiling_on_sc=False)


def _seg_body(tab2, idx_a, idx_b, dst_i, z32,
              agg_out,
              gi_v, di_v, rows_v, agg_sh, sem):
    c = lax.axis_index("c")
    s = lax.axis_index("s")
    pltpu.sync_copy(z32, agg_sh.at[pl.ds(s * NROW, NROW)])
    plsc.subcore_barrier()

    def run(idx_i):
        @pl.loop(0, SEG_IT)
        def _(g):
            r0 = s * RPT + g * SEG_CH
            pltpu.sync_copy(idx_i.at[pl.ds(r0, SEG_CH)], gi_v)
            pltpu.sync_copy(dst_i.at[pl.ds(r0, SEG_CH)], di_v)
            descs = [pltpu.async_copy(tab2.at[gi_v.at[j]],
                                      rows_v.at[pl.ds(j * CW, CW)], sem)
                     for j in range(SEG_CH)]
            for d in descs:
                d.wait()
            for j in range(SEG_CH):
                pltpu.sync_copy(rows_v.at[pl.ds(j * CW, CW)],
                                agg_sh.at[di_v.at[j]], add=True)

    @pl.when(c == 0)
    def _():
        run(idx_a)

    @pl.when(c == 1)
    def _():
        run(idx_b)

    plsc.subcore_barrier()
    pltpu.sync_copy(agg_sh.at[pl.ds(s * NROW, NROW)],
                    agg_out.at[c, pl.ds(s * NROW, NROW)])


_seg_call = functools.partial(
    pl.kernel, _seg_body,
    out_type=jax.ShapeDtypeStruct((NC, NUP, 32), f32),
    mesh=_mesh,
    compiler_params=_sc_params,
    scratch_types=[
        pltpu.VMEM((SEG_CH, CW), i32),
        pltpu.VMEM((SEG_CH, CW), i32),
        pltpu.VMEM((SEG_CH * CW, 32), f32),
        pltpu.VMEM_SHARED((NUP, 32), f32),
        pltpu.SemaphoreType.DMA,
    ],
)()


DEG_RPW = R // (NC * NS)
DEG_CH = 4
DEG_IT = DEG_RPW // DEG_CH


def _deg_body(dst_s, dst_r, dst_d, z8, one8,
              deg_s_out, deg_r_out, deg_d_out,
              di_v, ones_v, sh_s, sh_r, sh_d):
    c = lax.axis_index("c")
    s = lax.axis_index("s")
    w = s * NC + c
    pltpu.sync_copy(z8, sh_s.at[pl.ds(s * NROW, NROW)])
    pltpu.sync_copy(z8, sh_r.at[pl.ds(s * NROW, NROW)])
    pltpu.sync_copy(z8, sh_d.at[pl.ds(s * NROW, NROW)])
    pltpu.sync_copy(one8, ones_v)
    plsc.subcore_barrier()

    for dst_i, sh in ((dst_s, sh_s), (dst_r, sh_r), (dst_d, sh_d)):
        @pl.loop(0, DEG_IT)
        def _(g, dst_i=dst_i, sh=sh):
            r0 = w * DEG_RPW + g * DEG_CH
            pltpu.sync_copy(dst_i.at[pl.ds(r0, DEG_CH)], di_v)
            for j in range(DEG_CH):
                pltpu.sync_copy(ones_v, sh.at[di_v.at[j]], add=True)

    plsc.subcore_barrier()
    sl = pl.ds(s * NROW, NROW)
    pltpu.sync_copy(sh_s.at[sl], deg_s_out.at[c, sl])
    pltpu.sync_copy(sh_r.at[sl], deg_r_out.at[c, sl])
    pltpu.sync_copy(sh_d.at[sl], deg_d_out.at[c, sl])


_deg_call = functools.partial(
    pl.kernel, _deg_body,
    out_type=(jax.ShapeDtypeStruct((NC, NUP, 8), f32),) * 3,
    mesh=_mesh,
    compiler_params=_sc_params,
    scratch_types=[
        pltpu.VMEM((DEG_CH, CW), i32),
        pltpu.VMEM((CW, 8), f32),
        pltpu.VMEM_SHARED((NUP, 8), f32),
        pltpu.VMEM_SHARED((NUP, 8), f32),
        pltpu.VMEM_SHARED((NUP, 8), f32),
    ],
)()


GAT_CH = 8
GAT_RPT = R // (NC * NS)
GAT_IT = GAT_RPT // GAT_CH


def _gat_body(xu, xv, src_i, dst_i, xug_out, xvg_out, gi_v, rows_v, sem):
    c = lax.axis_index("c")
    s = lax.axis_index("s")
    w = s * NC + c

    @pl.loop(0, GAT_IT)
    def _(g):
        r0 = w * GAT_RPT + g * GAT_CH
        e0 = r0 * CW
        pltpu.sync_copy(src_i.at[pl.ds(r0, GAT_CH)], gi_v)
        descs = [pltpu.async_copy(xu.at[gi_v.at[j]],
                                  rows_v.at[pl.ds(j * CW, CW)], sem)
                 for j in range(GAT_CH)]
        for d in descs:
            d.wait()
        pltpu.sync_copy(rows_v, xug_out.at[pl.ds(e0, GAT_CH * CW)])
        pltpu.sync_copy(dst_i.at[pl.ds(r0, GAT_CH)], gi_v)
        descs = [pltpu.async_copy(xv.at[gi_v.at[j]],
                                  rows_v.at[pl.ds(j * CW, CW)], sem)
                 for j in range(GAT_CH)]
        for d in descs:
            d.wait()
        pltpu.sync_copy(rows_v, xvg_out.at[pl.ds(e0, GAT_CH * CW)])


_gat_call = functools.partial(
    pl.kernel, _gat_body,
    out_type=(jax.ShapeDtypeStruct((EP, D), f32),
              jax.ShapeDtypeStruct((EP, D), f32)),
    mesh=_mesh,
    compiler_params=_sc_params,
    scratch_types=[
        pltpu.VMEM((GAT_CH, CW), i32),
        pltpu.VMEM((GAT_CH * CW, D), f32),
        pltpu.SemaphoreType.DMA,
    ],
)()


BN_ = 1000


def _dense_body(hu, hv, agg_s, agg_d, agg_r, deg_s, deg_d, deg_r,
                sss, ssn, sds, sdn, srs, srn, ssb, sdb, srb,
                w1u, b1u, w2u, b2u, w1v, b1v, w2v, b2v,
                g1, be1, g2, be2,
                xu_out, xv_out):
    k = f32(1.0) / jnp.sqrt(f32(1.0) + f32(EPS))
    dot = functools.partial(jnp.dot, preferred_element_type=f32)

    ds_ = deg_s[...]
    dd_ = deg_d[...]
    dr_ = deg_r[...]
    degs = jnp.maximum(ds_[0, :, 0:1] + ds_[1, :, 0:1], f32(1.0))
    degd = jnp.maximum(dd_[0, :, 0:1] + dd_[1, :, 0:1], f32(1.0))
    degr = jnp.maximum(dr_[0, :, 0:1] + dr_[1, :, 0:1], f32(1.0))

    ssn_ = ssn[...]
    sdn_ = sdn[...]
    srn_ = srn[...]
    ts = (dot(agg_s[0], ssn_[:32, :]) + dot(agg_s[1], ssn_[32:, :])) / degs
    td = (dot(agg_d[0], sdn_[:32, :]) + dot(agg_d[1], sdn_[32:, :])) / degd
    tr = (dot(agg_r[0], srn_[:32, :]) + dot(agg_r[1], srn_[32:, :])) / degr

    emb_u = dot(hu[...], sss[...] + sds[...]) + ts + td + ssb[...] + sdb[...]
    a = dot(emb_u, w1u[...]) + b1u[...]
    a = jnp.maximum(a * (g1[...] * k) + be1[...], f32(0.0))
    xu_out[...] = dot(a, w2u[...]) + b2u[...]

    emb_v = dot(hv[...], srs[...]) + tr + srb[...]
    b = dot(emb_v, w1v[...]) + b1v[...]
    b = jnp.maximum(b * (g2[...] * k) + be2[...], f32(0.0))
    xv_out[...] = dot(b, w2v[...]) + b2v[...]


def _dense_call(hu, hv, agg_s, agg_d, agg_r, deg_s, deg_d, deg_r, *weights):
    n_blk = lambda i: (i, 0)
    a_blk = lambda i: (0, i, 0)
    w_blk = lambda i: (0, 0)
    in_specs = (
        [pl.BlockSpec((BN_, D), n_blk), pl.BlockSpec((BN_, D), n_blk)]
        + [pl.BlockSpec((NC, BN_, 32), a_blk)] * 3
        + [pl.BlockSpec((NC, BN_, 8), a_blk)] * 3
        + [pl.BlockSpec(w.shape, w_blk) for w in weights]
    )
    return pl.pallas_call(
        _dense_body,
        grid=(NU // BN_,),
        in_specs=in_specs,
        out_specs=(pl.BlockSpec((BN_, D), n_blk), pl.BlockSpec((BN_, D), n_blk)),
        out_shape=(jax.ShapeDtypeStruct((NU, D), f32),
                   jax.ShapeDtypeStruct((NI, D), f32)),
    )(hu, hv, agg_s, agg_d, agg_r, deg_s, deg_d, deg_r, *weights)


BE_ = 6400


def _head_body(xug, xvg, w1, b1, g3, be3, w2, b2, g4, be4, w3r, b3s, out):
    k = f32(1.0) / jnp.sqrt(f32(1.0) + f32(EPS))
    dot = functools.partial(jnp.dot, preferred_element_type=f32)
    p = xug[...] * xvg[...]
    t = dot(p, w1[...]) + b1[...]
    t = jnp.maximum(t * (g3[...] * k) + be3[...], f32(0.0))
    t = dot(t, w2[...]) + b2[...]
    t = jnp.maximum(t * (g4[...] * k) + be4[...], f32(0.0))
    out[...] = jnp.sum(t * w3r[...], axis=1, keepdims=True) + b3s[...]


def _head_call(xug, xvg, *weights):
    e_blk = lambda i: (i, 0)
    w_blk = lambda i: (0, 0)
    return pl.pallas_call(
        _head_body,
        grid=(EP // BE_,),
        in_specs=([pl.BlockSpec((BE_, D), e_blk), pl.BlockSpec((BE_, D), e_blk)]
                  + [pl.BlockSpec(w.shape, w_blk) for w in weights]),
        out_specs=pl.BlockSpec((BE_, 1), e_blk),
        out_shape=jax.ShapeDtypeStruct((EP, 1), f32),
    )(xug, xvg, *weights)


def _edge_prep(edges):
    src = edges[0].astype(i32)
    dst = edges[1].astype(i32)
    npad = EP - E
    src = jnp.concatenate([src, jnp.zeros((npad,), i32)])
    dst = jnp.concatenate([dst, jnp.full((npad,), PAD_DST, i32)])
    return ((2 * src).reshape(R, CW), (2 * src + 1).reshape(R, CW),
            dst.reshape(R, CW), src.reshape(R, CW))


def kernel(social_edges, rates_edges, rated_edges, u2e_w, v2e_w,
           sage_social_ws, sage_social_wn, sage_social_b,
           sage_rates_ws, sage_rates_wn, sage_rates_b,
           sage_rated_ws, sage_rated_wn, sage_rated_b,
           wur1_w, wur1_b, wur2_w, wur2_b,
           wvr1_w, wvr1_b, wvr2_w, wvr2_b,
           wuv1_w, wuv1_b, wuv2_w, wuv2_b, wuv3_w, wuv3_b,
           bn1_g, bn1_b, bn2_g, bn2_b, bn3_g, bn3_b, bn4_g, bn4_b):
    sa, sb, sd, _ = _edge_prep(social_edges)
    ra, rb, rd, rs = _edge_prep(rates_edges)
    da, db, dd, _ = _edge_prep(rated_edges)

    tab_u = u2e_w.reshape(2 * NU, 32)
    tab_v = v2e_w.reshape(2 * NI, 32)
    z32 = jnp.zeros((NROW, 32), f32)
    z8 = jnp.zeros((NROW, 8), f32)
    one8 = jnp.ones((CW, 8), f32)

    agg_s = _seg_call(tab_u, sa, sb, sd, z32)
    agg_r = _seg_call(tab_u, ra, rb, rd, z32)
    agg_d = _seg_call(tab_v, da, db, dd, z32)
    deg_s, deg_r, deg_d = _deg_call(sd, rd, dd, z8, one8)

    row = lambda x: x.reshape(1, -1)
    x_u, x_v = _dense_call(
        u2e_w, v2e_w, agg_s, agg_d, agg_r, deg_s, deg_d, deg_r,
        sage_social_ws, sage_social_wn, sage_rated_ws, sage_rated_wn,
        sage_rates_ws, sage_rates_wn,
        row(sage_social_b), row(sage_rated_b), row(sage_rates_b),
        wur1_w, row(wur1_b), wur2_w, row(wur2_b),
        wvr1_w, row(wvr1_b), wvr2_w, row(wvr2_b),
        row(bn1_g), row(bn1_b), row(bn2_g), row(bn2_b))

    xug, xvg = _gat_call(x_u, x_v, rs, rd)

    scores2 = _head_call(
        xug, xvg, wuv1_w, row(wuv1_b), row(bn3_g), row(bn3_b),
        wuv2_w, row(wuv2_b), row(bn4_g), row(bn4_b),
        wuv3_w.reshape(1, 16), row(wuv3_b))
    return scores2[:E, 0]


# speedup vs baseline: 3.7891x; 1.1141x over previous
import functools

import jax
import jax.numpy as jnp
from jax import lax
from jax.experimental import pallas as pl
from jax.experimental.pallas import tpu as pltpu
from jax.experimental.pallas import tpu_sc as plsc

NU = 50000
NI = 50000
D = 64
E = 800000
EPS = 1e-5

NC = 2
NS = 16
CW = 128
EP = 819200
R = EP // CW
RPT = R // NS
SEG_CH = 2
SEG_IT = RPT // SEG_CH
NROW = 3128
NUP = NROW * NS
PAD_DST = NU

f32 = jnp.float32
i32 = jnp.int32

_mesh = plsc.VectorSubcoreMesh(core_axis_name="c", subcore_axis_name="s",
                               num_cores=NC, num_subcores=NS)
_sc_params = pltpu.CompilerParams(use_tc_tiling_on_sc=False)


def _seg_body(tab2, idx_a, idx_b, dst_i, z32,
              agg_out,
              gi0, gi1, di0, di1, rows0, rows1, agg_sh, sem0, sem1):
    c = lax.axis_index("c")
    s = lax.axis_index("s")
    pltpu.sync_copy(z32, agg_sh.at[pl.ds(s * NROW, NROW)])
    plsc.subcore_barrier()

    slots = ((gi0, di0, rows0, sem0), (gi1, di1, rows1, sem1))

    def run(idx_i):
        def fire(g, slot):
            gi, di, rows, sem = slots[slot]
            r0 = s * RPT + g * SEG_CH
            pltpu.sync_copy(idx_i.at[pl.ds(r0, SEG_CH)], gi)
            pltpu.sync_copy(dst_i.at[pl.ds(r0, SEG_CH)], di)
            for j in range(SEG_CH):
                pltpu.async_copy(tab2.at[gi.at[j]],
                                 rows.at[pl.ds(j * CW, CW)], sem)

        def drain_scatter(slot):
            gi, di, rows, sem = slots[slot]
            for j in range(SEG_CH):
                pltpu.make_async_copy(tab2.at[gi.at[j]],
                                      rows.at[pl.ds(j * CW, CW)], sem).wait()
            for j in range(SEG_CH):
                pltpu.sync_copy(rows.at[pl.ds(j * CW, CW)],
                                agg_sh.at[di.at[j]], add=True)

        fire(0, 0)

        @pl.loop(0, SEG_IT, step=2)
        def _(g):
            fire(g + 1, 1)
            drain_scatter(0)

            @pl.when(g + 2 < SEG_IT)
            def _():
                fire(g + 2, 0)

            drain_scatter(1)

    @pl.when(c == 0)
    def _():
        run(idx_a)

    @pl.when(c == 1)
    def _():
        run(idx_b)

    plsc.subcore_barrier()
    pltpu.sync_copy(agg_sh.at[pl.ds(s * NROW, NROW)],
                    agg_out.at[c, pl.ds(s * NROW, NROW)])


_seg_call = functools.partial(
    pl.kernel, _seg_body,
    out_type=jax.ShapeDtypeStruct((NC, NUP, 32), f32),
    mesh=_mesh,
    compiler_params=_sc_params,
    scratch_types=[
        pltpu.VMEM((SEG_CH, CW), i32),
        pltpu.VMEM((SEG_CH, CW), i32),
        pltpu.VMEM((SEG_CH, CW), i32),
        pltpu.VMEM((SEG_CH, CW), i32),
        pltpu.VMEM((SEG_CH * CW, 32), f32),
        pltpu.VMEM((SEG_CH * CW, 32), f32),
        pltpu.VMEM_SHARED((NUP, 32), f32),
        pltpu.SemaphoreType.DMA,
        pltpu.SemaphoreType.DMA,
    ],
)()


DEG_RPW = R // (NC * NS)
DEG_CH = 4
DEG_IT = DEG_RPW // DEG_CH


def _deg_body(dst_s, dst_r, dst_d, z8, one8,
              deg_s_out, deg_r_out, deg_d_out,
              di_v, ones_v, sh_s, sh_r, sh_d):
    c = lax.axis_index("c")
    s = lax.axis_index("s")
    w = s * NC + c
    pltpu.sync_copy(z8, sh_s.at[pl.ds(s * NROW, NROW)])
    pltpu.sync_copy(z8, sh_r.at[pl.ds(s * NROW, NROW)])
    pltpu.sync_copy(z8, sh_d.at[pl.ds(s * NROW, NROW)])
    pltpu.sync_copy(one8, ones_v)
    plsc.subcore_barrier()

    for dst_i, sh in ((dst_s, sh_s), (dst_r, sh_r), (dst_d, sh_d)):
        @pl.loop(0, DEG_IT)
        def _(g, dst_i=dst_i, sh=sh):
            r0 = w * DEG_RPW + g * DEG_CH
            pltpu.sync_copy(dst_i.at[pl.ds(r0, DEG_CH)], di_v)
            for j in range(DEG_CH):
                pltpu.sync_copy(ones_v, sh.at[di_v.at[j]], add=True)

    plsc.subcore_barrier()
    sl = pl.ds(s * NROW, NROW)
    pltpu.sync_copy(sh_s.at[sl], deg_s_out.at[c, sl])
    pltpu.sync_copy(sh_r.at[sl], deg_r_out.at[c, sl])
    pltpu.sync_copy(sh_d.at[sl], deg_d_out.at[c, sl])


_deg_call = functools.partial(
    pl.kernel, _deg_body,
    out_type=(jax.ShapeDtypeStruct((NC, NUP, 8), f32),) * 3,
    mesh=_mesh,
    compiler_params=_sc_params,
    scratch_types=[
        pltpu.VMEM((DEG_CH, CW), i32),
        pltpu.VMEM((CW, 8), f32),
        pltpu.VMEM_SHARED((NUP, 8), f32),
        pltpu.VMEM_SHARED((NUP, 8), f32),
        pltpu.VMEM_SHARED((NUP, 8), f32),
    ],
)()


GAT_CH = 2
GAT_RPT = R // (NC * NS)
GAT_IT = GAT_RPT // GAT_CH


def _gat_body(xu, xv, src_i, dst_i, xug_out, xvg_out,
              si0, si1, di0, di1, ra0, ra1, rb0, rb1, sem0, sem1):
    c = lax.axis_index("c")
    s = lax.axis_index("s")
    w = s * NC + c
    slots = ((si0, di0, ra0, rb0, sem0), (si1, di1, ra1, rb1, sem1))

    def fire(g, slot):
        si, di, ra, rb, sem = slots[slot]
        r0 = w * GAT_RPT + g * GAT_CH
        pltpu.sync_copy(src_i.at[pl.ds(r0, GAT_CH)], si)
        pltpu.sync_copy(dst_i.at[pl.ds(r0, GAT_CH)], di)
        for j in range(GAT_CH):
            pltpu.async_copy(xu.at[si.at[j]], ra.at[pl.ds(j * CW, CW)], sem)
            pltpu.async_copy(xv.at[di.at[j]], rb.at[pl.ds(j * CW, CW)], sem)

    def drain_write(g, slot):
        si, di, ra, rb, sem = slots[slot]
        r0 = w * GAT_RPT + g * GAT_CH
        e0 = r0 * CW
        for j in range(GAT_CH):
            pltpu.make_async_copy(xu.at[si.at[j]],
                                  ra.at[pl.ds(j * CW, CW)], sem).wait()
            pltpu.make_async_copy(xv.at[di.at[j]],
                                  rb.at[pl.ds(j * CW, CW)], sem).wait()
        pltpu.sync_copy(ra, xug_out.at[pl.ds(e0, GAT_CH * CW)])
        pltpu.sync_copy(rb, xvg_out.at[pl.ds(e0, GAT_CH * CW)])

    fire(0, 0)

    @pl.loop(0, GAT_IT, step=2)
    def _(g):
        fire(g + 1, 1)
        drain_write(g, 0)

        @pl.when(g + 2 < GAT_IT)
        def _():
            fire(g + 2, 0)

        drain_write(g + 1, 1)


_gat_call = functools.partial(
    pl.kernel, _gat_body,
    out_type=(jax.ShapeDtypeStruct((EP, D), f32),
              jax.ShapeDtypeStruct((EP, D), f32)),
    mesh=_mesh,
    compiler_params=_sc_params,
    scratch_types=[
        pltpu.VMEM((GAT_CH, CW), i32),
        pltpu.VMEM((GAT_CH, CW), i32),
        pltpu.VMEM((GAT_CH, CW), i32),
        pltpu.VMEM((GAT_CH, CW), i32),
        pltpu.VMEM((GAT_CH * CW, D), f32),
        pltpu.VMEM((GAT_CH * CW, D), f32),
        pltpu.VMEM((GAT_CH * CW, D), f32),
        pltpu.VMEM((GAT_CH * CW, D), f32),
        pltpu.SemaphoreType.DMA,
        pltpu.SemaphoreType.DMA,
    ],
)()


BN_ = 1000


def _dense_body(hu, hv, agg_s, agg_d, agg_r, deg_s, deg_d, deg_r,
                sss, ssn, sds, sdn, srs, srn, ssb, sdb, srb,
                w1u, b1u, w2u, b2u, w1v, b1v, w2v, b2v,
                g1, be1, g2, be2,
                xu_out, xv_out):
    k = f32(1.0) / jnp.sqrt(f32(1.0) + f32(EPS))
    dot = functools.partial(jnp.dot, preferred_element_type=f32)

    ds_ = deg_s[...]
    dd_ = deg_d[...]
    dr_ = deg_r[...]
    degs = jnp.maximum(ds_[0, :, 0:1] + ds_[1, :, 0:1], f32(1.0))
    degd = jnp.maximum(dd_[0, :, 0:1] + dd_[1, :, 0:1], f32(1.0))
    degr = jnp.maximum(dr_[0, :, 0:1] + dr_[1, :, 0:1], f32(1.0))

    ssn_ = ssn[...]
    sdn_ = sdn[...]
    srn_ = srn[...]
    ts = (dot(agg_s[0], ssn_[:32, :]) + dot(agg_s[1], ssn_[32:, :])) / degs
    td = (dot(agg_d[0], sdn_[:32, :]) + dot(agg_d[1], sdn_[32:, :])) / degd
    tr = (dot(agg_r[0], srn_[:32, :]) + dot(agg_r[1], srn_[32:, :])) / degr

    emb_u = dot(hu[...], sss[...] + sds[...]) + ts + td + ssb[...] + sdb[...]
    a = dot(emb_u, w1u[...]) + b1u[...]
    a = jnp.maximum(a * (g1[...] * k) + be1[...], f32(0.0))
    xu_out[...] = dot(a, w2u[...]) + b2u[...]

    emb_v = dot(hv[...], srs[...]) + tr + srb[...]
    b = dot(emb_v, w1v[...]) + b1v[...]
    b = jnp.maximum(b * (g2[...] * k) + be2[...], f32(0.0))
    xv_out[...] = dot(b, w2v[...]) + b2v[...]


def _dense_call(hu, hv, agg_s, agg_d, agg_r, deg_s, deg_d, deg_r, *weights):
    n_blk = lambda i: (i, 0)
    a_blk = lambda i: (0, i, 0)
    w_blk = lambda i: (0, 0)
    in_specs = (
        [pl.BlockSpec((BN_, D), n_blk), pl.BlockSpec((BN_, D), n_blk)]
        + [pl.BlockSpec((NC, BN_, 32), a_blk)] * 3
        + [pl.BlockSpec((NC, BN_, 8), a_blk)] * 3
        + [pl.BlockSpec(w.shape, w_blk) for w in weights]
    )
    return pl.pallas_call(
        _dense_body,
        grid=(NU // BN_,),
        in_specs=in_specs,
        out_specs=(pl.BlockSpec((BN_, D), n_blk), pl.BlockSpec((BN_, D), n_blk)),
        out_shape=(jax.ShapeDtypeStruct((NU, D), f32),
                   jax.ShapeDtypeStruct((NI, D), f32)),
    )(hu, hv, agg_s, agg_d, agg_r, deg_s, deg_d, deg_r, *weights)


BE_ = 6400


def _head_body(xug, xvg, w1, b1, g3, be3, w2, b2, g4, be4, w3r, b3s, out):
    k = f32(1.0) / jnp.sqrt(f32(1.0) + f32(EPS))
    dot = functools.partial(jnp.dot, preferred_element_type=f32)
    p = xug[...] * xvg[...]
    t = dot(p, w1[...]) + b1[...]
    t = jnp.maximum(t * (g3[...] * k) + be3[...], f32(0.0))
    t = dot(t, w2[...]) + b2[...]
    t = jnp.maximum(t * (g4[...] * k) + be4[...], f32(0.0))
    out[...] = jnp.sum(t * w3r[...], axis=1, keepdims=True) + b3s[...]


def _head_call(xug, xvg, *weights):
    e_blk = lambda i: (i, 0)
    w_blk = lambda i: (0, 0)
    return pl.pallas_call(
        _head_body,
        grid=(EP // BE_,),
        in_specs=([pl.BlockSpec((BE_, D), e_blk), pl.BlockSpec((BE_, D), e_blk)]
                  + [pl.BlockSpec(w.shape, w_blk) for w in weights]),
        out_specs=pl.BlockSpec((BE_, 1), e_blk),
        out_shape=jax.ShapeDtypeStruct((EP, 1), f32),
    )(xug, xvg, *weights)


def _edge_prep(edges):
    src = edges[0].astype(i32)
    dst = edges[1].astype(i32)
    npad = EP - E
    src = jnp.concatenate([src, jnp.zeros((npad,), i32)])
    dst = jnp.concatenate([dst, jnp.full((npad,), PAD_DST, i32)])
    return ((2 * src).reshape(R, CW), (2 * src + 1).reshape(R, CW),
            dst.reshape(R, CW), src.reshape(R, CW))


def kernel(social_edges, rates_edges, rated_edges, u2e_w, v2e_w,
           sage_social_ws, sage_social_wn, sage_social_b,
           sage_rates_ws, sage_rates_wn, sage_rates_b,
           sage_rated_ws, sage_rated_wn, sage_rated_b,
           wur1_w, wur1_b, wur2_w, wur2_b,
           wvr1_w, wvr1_b, wvr2_w, wvr2_b,
           wuv1_w, wuv1_b, wuv2_w, wuv2_b, wuv3_w, wuv3_b,
           bn1_g, bn1_b, bn2_g, bn2_b, bn3_g, bn3_b, bn4_g, bn4_b):
    sa, sb, sd, _ = _edge_prep(social_edges)
    ra, rb, rd, rs = _edge_prep(rates_edges)
    da, db, dd, _ = _edge_prep(rated_edges)

    tab_u = u2e_w.reshape(2 * NU, 32)
    tab_v = v2e_w.reshape(2 * NI, 32)
    z32 = jnp.zeros((NROW, 32), f32)
    z8 = jnp.zeros((NROW, 8), f32)
    one8 = jnp.ones((CW, 8), f32)

    agg_s = _seg_call(tab_u, sa, sb, sd, z32)
    agg_r = _seg_call(tab_u, ra, rb, rd, z32)
    agg_d = _seg_call(tab_v, da, db, dd, z32)
    deg_s, deg_r, deg_d = _deg_call(sd, rd, dd, z8, one8)

    row = lambda x: x.reshape(1, -1)
    x_u, x_v = _dense_call(
        u2e_w, v2e_w, agg_s, agg_d, agg_r, deg_s, deg_d, deg_r,
        sage_social_ws, sage_social_wn, sage_rated_ws, sage_rated_wn,
        sage_rates_ws, sage_rates_wn,
        row(sage_social_b), row(sage_rated_b), row(sage_rates_b),
        wur1_w, row(wur1_b), wur2_w, row(wur2_b),
        wvr1_w, row(wvr1_b), wvr2_w, row(wvr2_b),
        row(bn1_g), row(bn1_b), row(bn2_g), row(bn2_b))

    xug, xvg = _gat_call(x_u, x_v, rs, rd)

    scores2 = _head_call(
        xug, xvg, wuv1_w, row(wuv1_b), row(bn3_g), row(bn3_b),
        wuv2_w, row(wuv2_b), row(bn4_g), row(bn4_b),
        wuv3_w.reshape(1, 16), row(wuv3_b))
    return scores2[:E, 0]
